# baseline (device time: 519042 ns/iter reference)
import jax
import jax.numpy as jnp
from jax import lax
from jax.experimental import pallas as pl
from jax.experimental.pallas import tpu as pltpu

N_DEV = 8
S = 1024
H = 8
D = 128
SCALE = 0.08838834764831843
NEG = -1e9


def _attn_body(q_ref, k_ref, v_ref, out_ref,
               comm_ref, acc_ref, m_ref, l_ref, bias_ref,
               send_sems, recv_sems):
    my = lax.axis_index("i")
    left = lax.rem(my - 1 + N_DEV, N_DEV)
    right = lax.rem(my + 1, N_DEV)

    barrier_sem = pltpu.get_barrier_semaphore()
    for nbr in (left, right):
        pl.semaphore_signal(barrier_sem, inc=1, device_id=(nbr,),
                            device_id_type=pl.DeviceIdType.MESH)
    pl.semaphore_wait(barrier_sem, 2)

    row_r = (lax.broadcasted_iota(jnp.int32, (S, S), 0) // 64) % 4
    col_r = (lax.broadcasted_iota(jnp.int32, (S, S), 1) // 64) % 4
    bias_ref[:, :] = jnp.where(row_r == col_r, 0.0, NEG).astype(jnp.float32)

    m_ref[:, :, :] = jnp.full((H, S, 1), -1e30, jnp.float32)
    l_ref[:, :, :] = jnp.zeros((H, S, 1), jnp.float32)
    acc_ref[:, :, :] = jnp.zeros((H, S, D), jnp.float32)

    comm_ref[0, 0] = k_ref[:, :, :]
    comm_ref[0, 1] = v_ref[:, :, :]

    def chunk_update(slot):
        def head_body(h, carry):
            q_h = q_ref[h]
            k_h = comm_ref[slot, 0, h]
            v_h = comm_ref[slot, 1, h]
            s = lax.dot_general(
                q_h, k_h, (((1,), (1,)), ((), ())),
                preferred_element_type=jnp.float32,
            )
            s = s + bias_ref[:, :]
            m_old = m_ref[h]
            m_new = jnp.maximum(m_old, jnp.max(s, axis=1, keepdims=True))
            alpha = jnp.exp(m_old - m_new)
            p = jnp.exp(s - m_new)
            l_ref[h] = l_ref[h] * alpha + jnp.sum(p, axis=1, keepdims=True)
            pv = lax.dot_general(
                p.astype(jnp.bfloat16), v_h, (((1,), (0,)), ((), ())),
                preferred_element_type=jnp.float32,
            )
            acc_ref[h] = acc_ref[h] * alpha + pv
            m_ref[h] = m_new
            return carry
        lax.fori_loop(0, H, head_body, 0)

    chunk_update(0)

    for h in range(N_DEV - 1):
        send_slot = h % 2
        recv_slot = (h + 1) % 2
        rdma = pltpu.make_async_remote_copy(
            src_ref=comm_ref.at[send_slot],
            dst_ref=comm_ref.at[recv_slot],
            send_sem=send_sems.at[send_slot],
            recv_sem=recv_sems.at[recv_slot],
            device_id=(right,),
            device_id_type=pl.DeviceIdType.MESH,
        )
        rdma.start()
        rdma.wait()
        chunk_update(recv_slot)

    def final_body(h, carry):
        out_ref[h] = acc_ref[h] / l_ref[h]
        return carry
    lax.fori_loop(0, H, final_body, 0)


def kernel(x, Wq, K_ext, V_ext, Wo):
    x2 = x.reshape(S, H * D)
    q = (jnp.dot(x2, Wq) * SCALE).reshape(S, H, D)
    qh = q.transpose(1, 0, 2).astype(jnp.bfloat16)
    kh = K_ext[0].transpose(1, 0, 2).astype(jnp.bfloat16)
    vh = V_ext[0].transpose(1, 0, 2).astype(jnp.bfloat16)

    ctx = pl.pallas_call(
        _attn_body,
        out_shape=jax.ShapeDtypeStruct((H, S, D), jnp.float32),
        in_specs=[
            pl.BlockSpec(memory_space=pltpu.VMEM),
            pl.BlockSpec(memory_space=pltpu.VMEM),
            pl.BlockSpec(memory_space=pltpu.VMEM),
        ],
        out_specs=pl.BlockSpec(memory_space=pltpu.VMEM),
        scratch_shapes=[
            pltpu.VMEM((2, 2, H, S, D), jnp.bfloat16),
            pltpu.VMEM((H, S, D), jnp.float32),
            pltpu.VMEM((H, S, 1), jnp.float32),
            pltpu.VMEM((H, S, 1), jnp.float32),
            pltpu.VMEM((S, S), jnp.float32),
            pltpu.SemaphoreType.DMA((2,)),
            pltpu.SemaphoreType.DMA((2,)),
        ],
        compiler_params=pltpu.CompilerParams(collective_id=0),
    )(qh, kh, vh)

    ctx2 = ctx.transpose(1, 0, 2).reshape(S, H * D)
    return jnp.dot(ctx2, Wo).reshape(1, S, H * D)


# device time: 370943 ns/iter; 1.3993x vs baseline; 1.3993x over previous
import jax
import jax.numpy as jnp
from jax import lax
from jax.experimental import pallas as pl
from jax.experimental.pallas import tpu as pltpu

N_DEV = 8
S = 1024
H = 8
D = 128
SCALE = 0.08838834764831843
NEG = -1e9


def _attn_body(q_ref, k_ref, v_ref, out_ref,
               comm_ref, acc_ref, m_ref, l_ref, bias_ref,
               send_sems, recv_sems):
    my = lax.axis_index("i")
    left = lax.rem(my - 1 + N_DEV, N_DEV)
    right = lax.rem(my + 1, N_DEV)

    barrier_sem = pltpu.get_barrier_semaphore()
    for nbr in (left, right):
        pl.semaphore_signal(barrier_sem, inc=1, device_id=(nbr,),
                            device_id_type=pl.DeviceIdType.MESH)
    pl.semaphore_wait(barrier_sem, 2)

    row_r = (lax.broadcasted_iota(jnp.int32, (S, S), 0) // 64) % 4
    col_r = (lax.broadcasted_iota(jnp.int32, (S, S), 1) // 64) % 4
    bias_ref[:, :] = jnp.where(row_r == col_r, 0.0, NEG).astype(jnp.float32)

    m_ref[:, :, :] = jnp.full((H, S, 1), -1e30, jnp.float32)
    l_ref[:, :, :] = jnp.zeros((H, S, 1), jnp.float32)
    acc_ref[:, :, :] = jnp.zeros((H, S, D), jnp.float32)

    comm_ref[0, 0] = k_ref[:, :, :]
    comm_ref[0, 1] = v_ref[:, :, :]

    def chunk_update(slot):
        def head_body(h, carry):
            q_h = q_ref[h]
            k_h = comm_ref[slot, 0, h]
            v_h = comm_ref[slot, 1, h]
            s = lax.dot_general(
                q_h, k_h, (((1,), (1,)), ((), ())),
                preferred_element_type=jnp.float32,
            )
            s = s + bias_ref[:, :]
            m_old = m_ref[h]
            m_new = jnp.maximum(m_old, jnp.max(s, axis=1, keepdims=True))
            alpha = jnp.exp(m_old - m_new)
            p = jnp.exp(s - m_new)
            l_ref[h] = l_ref[h] * alpha + jnp.sum(p, axis=1, keepdims=True)
            pv = lax.dot_general(
                p.astype(jnp.bfloat16), v_h, (((1,), (0,)), ((), ())),
                preferred_element_type=jnp.float32,
            )
            acc_ref[h] = acc_ref[h] * alpha + pv
            m_ref[h] = m_new
            return carry
        lax.fori_loop(0, H, head_body, 0)

    for h in range(N_DEV - 1):
        send_slot = h % 2
        recv_slot = (h + 1) % 2
        rdma = pltpu.make_async_remote_copy(
            src_ref=comm_ref.at[send_slot],
            dst_ref=comm_ref.at[recv_slot],
            send_sem=send_sems.at[send_slot],
            recv_sem=recv_sems.at[recv_slot],
            device_id=(right,),
            device_id_type=pl.DeviceIdType.MESH,
        )
        rdma.start()
        chunk_update(send_slot)
        rdma.wait()
    chunk_update((N_DEV - 1) % 2)

    def final_body(h, carry):
        out_ref[h] = acc_ref[h] / l_ref[h]
        return carry
    lax.fori_loop(0, H, final_body, 0)


def kernel(x, Wq, K_ext, V_ext, Wo):
    x2 = x.reshape(S, H * D)
    q = (jnp.dot(x2, Wq) * SCALE).reshape(S, H, D)
    qh = q.transpose(1, 0, 2).astype(jnp.bfloat16)
    kh = K_ext[0].transpose(1, 0, 2).astype(jnp.bfloat16)
    vh = V_ext[0].transpose(1, 0, 2).astype(jnp.bfloat16)

    ctx = pl.pallas_call(
        _attn_body,
        out_shape=jax.ShapeDtypeStruct((H, S, D), jnp.float32),
        in_specs=[
            pl.BlockSpec(memory_space=pltpu.VMEM),
            pl.BlockSpec(memory_space=pltpu.VMEM),
            pl.BlockSpec(memory_space=pltpu.VMEM),
        ],
        out_specs=pl.BlockSpec(memory_space=pltpu.VMEM),
        scratch_shapes=[
            pltpu.VMEM((2, 2, H, S, D), jnp.bfloat16),
            pltpu.VMEM((H, S, D), jnp.float32),
            pltpu.VMEM((H, S, 1), jnp.float32),
            pltpu.VMEM((H, S, 1), jnp.float32),
            pltpu.VMEM((S, S), jnp.float32),
            pltpu.SemaphoreType.DMA((2,)),
            pltpu.SemaphoreType.DMA((2,)),
        ],
        compiler_params=pltpu.CompilerParams(collective_id=0),
    )(qh, kh, vh)

    ctx2 = ctx.transpose(1, 0, 2).reshape(S, H * D)
    return jnp.dot(ctx2, Wo).reshape(1, S, H * D)


# device time: 243555 ns/iter; 2.1311x vs baseline; 1.5230x over previous
import jax
import jax.numpy as jnp
from jax import lax
from jax.experimental import pallas as pl
from jax.experimental.pallas import tpu as pltpu

N_DEV = 8
S = 1024
H = 8
D = 128
R = 4
RS = S // R
SCALE = 0.08838834764831843


def _attn_body(q_ref, k_ref, v_ref, out_ref,
               commR_ref, commL_ref, acc_ref, m_ref, l_ref,
               sendR_sems, recvR_sems, sendL_sems, recvL_sems):
    my = lax.axis_index("i")
    left = lax.rem(my - 1 + N_DEV, N_DEV)
    right = lax.rem(my + 1, N_DEV)

    barrier_sem = pltpu.get_barrier_semaphore()
    for nbr in (left, right):
        pl.semaphore_signal(barrier_sem, inc=1, device_id=(nbr,),
                            device_id_type=pl.DeviceIdType.MESH)
    pl.semaphore_wait(barrier_sem, 2)

    m_ref[:, :, :] = jnp.full((H, S, 1), -1e30, jnp.float32)
    l_ref[:, :, :] = jnp.zeros((H, S, 1), jnp.float32)
    acc_ref[:, :, :] = jnp.zeros((H, S, D), jnp.float32)

    commR_ref[0, 0] = k_ref[:, :, :]
    commR_ref[0, 1] = v_ref[:, :, :]
    commL_ref[0, 0] = k_ref[:, :, :]
    commL_ref[0, 1] = v_ref[:, :, :]

    def chunk_update(comm_ref, slot):
        def head_body(h, carry):
            for r in range(R):
                rs = pl.ds(r * RS, RS)
                q_h = q_ref[h, rs]
                k_h = comm_ref[slot, 0, h, rs]
                v_h = comm_ref[slot, 1, h, rs]
                s = lax.dot_general(
                    q_h, k_h, (((1,), (1,)), ((), ())),
                    preferred_element_type=jnp.float32,
                )
                m_old = m_ref[h, rs]
                m_new = jnp.maximum(m_old, jnp.max(s, axis=1, keepdims=True))
                alpha = jnp.exp(m_old - m_new)
                p = jnp.exp(s - m_new)
                l_ref[h, rs] = l_ref[h, rs] * alpha + jnp.sum(
                    p, axis=1, keepdims=True)
                pv = lax.dot_general(
                    p.astype(jnp.bfloat16), v_h, (((1,), (0,)), ((), ())),
                    preferred_element_type=jnp.float32,
                )
                acc_ref[h, rs] = acc_ref[h, rs] * alpha + pv
                m_ref[h, rs] = m_new
            return carry
        lax.fori_loop(0, H, head_body, 0)

    R_HOPS = N_DEV // 2
    L_HOPS = N_DEV - 1 - R_HOPS

    def start_hop(h, comm_ref, send_sems, recv_sems, dst):
        rdma = pltpu.make_async_remote_copy(
            src_ref=comm_ref.at[h % 3],
            dst_ref=comm_ref.at[(h + 1) % 3],
            send_sem=send_sems.at[h % 3],
            recv_sem=recv_sems.at[(h + 1) % 3],
            device_id=(dst,),
            device_id_type=pl.DeviceIdType.MESH,
        )
        rdma.start()
        return rdma

    for h in range(R_HOPS):
        rR = start_hop(h, commR_ref, sendR_sems, recvR_sems, right)
        rL = None
        if h < L_HOPS:
            rL = start_hop(h, commL_ref, sendL_sems, recvL_sems, left)
        chunk_update(commR_ref, h % 3)
        if h > 0:
            chunk_update(commL_ref, h % 3)
        rR.wait()
        if rL is not None:
            rL.wait()
    chunk_update(commR_ref, R_HOPS % 3)

    def final_body(h, carry):
        out_ref[h] = acc_ref[h] / l_ref[h]
        return carry
    lax.fori_loop(0, H, final_body, 0)


def _perm(a):
    return a.reshape(R, R, 64, *a.shape[1:]).swapaxes(0, 1).reshape(a.shape)


def kernel(x, Wq, K_ext, V_ext, Wo):
    x2 = x.reshape(S, H * D)
    q = (jnp.dot(x2, Wq) * SCALE).reshape(S, H, D)
    qh = _perm(q).transpose(1, 0, 2).astype(jnp.bfloat16)
    kh = _perm(K_ext[0]).transpose(1, 0, 2).astype(jnp.bfloat16)
    vh = _perm(V_ext[0]).transpose(1, 0, 2).astype(jnp.bfloat16)

    ctx = pl.pallas_call(
        _attn_body,
        out_shape=jax.ShapeDtypeStruct((H, S, D), jnp.float32),
        in_specs=[
            pl.BlockSpec(memory_space=pltpu.VMEM),
            pl.BlockSpec(memory_space=pltpu.VMEM),
            pl.BlockSpec(memory_space=pltpu.VMEM),
        ],
        out_specs=pl.BlockSpec(memory_space=pltpu.VMEM),
        scratch_shapes=[
            pltpu.VMEM((3, 2, H, S, D), jnp.bfloat16),
            pltpu.VMEM((3, 2, H, S, D), jnp.bfloat16),
            pltpu.VMEM((H, S, D), jnp.float32),
            pltpu.VMEM((H, S, 1), jnp.float32),
            pltpu.VMEM((H, S, 1), jnp.float32),
            pltpu.SemaphoreType.DMA((3,)),
            pltpu.SemaphoreType.DMA((3,)),
            pltpu.SemaphoreType.DMA((3,)),
            pltpu.SemaphoreType.DMA((3,)),
        ],
        compiler_params=pltpu.CompilerParams(
            collective_id=0,
            vmem_limit_bytes=56 * 1024 * 1024,
        ),
    )(qh, kh, vh)

    ctx2 = _perm(ctx.transpose(1, 0, 2)).reshape(S, H * D)
    return jnp.dot(ctx2, Wo).reshape(1, S, H * D)


# device time: 235188 ns/iter; 2.2069x vs baseline; 1.0356x over previous
import jax
import jax.numpy as jnp
from jax import lax
from jax.experimental import pallas as pl
from jax.experimental.pallas import tpu as pltpu

N_DEV = 8
S = 1024
H = 8
D = 128
R = 4
RS = S // R
SCALE = 0.08838834764831843


def _attn_body(q_ref, k_ref, v_ref, out_ref,
               commR_ref, commL_ref, acc_ref, m_ref, l_ref,
               sendR_sems, recvR_sems, sendL_sems, recvL_sems):
    my = lax.axis_index("i")
    left = lax.rem(my - 1 + N_DEV, N_DEV)
    right = lax.rem(my + 1, N_DEV)

    barrier_sem = pltpu.get_barrier_semaphore()
    for nbr in (left, right):
        pl.semaphore_signal(barrier_sem, inc=1, device_id=(nbr,),
                            device_id_type=pl.DeviceIdType.MESH)
    pl.semaphore_wait(barrier_sem, 2)

    m_ref[:, :, :] = jnp.full((H, S, 1), -1e30, jnp.float32)
    l_ref[:, :, :] = jnp.zeros((H, S, 1), jnp.float32)
    acc_ref[:, :, :] = jnp.zeros((H, S, D), jnp.float32)

    commR_ref[0, 0] = k_ref[:, :, :]
    commR_ref[0, 1] = v_ref[:, :, :]

    def chunk_update(comm_ref, slot, lo=0, hi=H):
        def head_body(h, carry):
            for r in range(R):
                rs = pl.ds(r * RS, RS)
                q_h = q_ref[h, rs]
                k_h = comm_ref[slot, 0, h, rs]
                v_h = comm_ref[slot, 1, h, rs]
                s = lax.dot_general(
                    q_h, k_h, (((1,), (1,)), ((), ())),
                    preferred_element_type=jnp.float32,
                )
                m_old = m_ref[h, rs]
                m_new = jnp.maximum(m_old, jnp.max(s, axis=1, keepdims=True))
                alpha = jnp.exp(m_old - m_new)
                p = jnp.exp(s - m_new)
                l_ref[h, rs] = l_ref[h, rs] * alpha + jnp.sum(
                    p, axis=1, keepdims=True)
                pv = lax.dot_general(
                    p.astype(jnp.bfloat16), v_h, (((1,), (0,)), ((), ())),
                    preferred_element_type=jnp.float32,
                )
                acc_ref[h, rs] = acc_ref[h, rs] * alpha + pv
                m_ref[h, rs] = m_new
            return carry
        lax.fori_loop(lo, hi, head_body, 0)

    HH = H // 2

    def mk_rdma(src, dst, ssem, rsem, dev):
        return pltpu.make_async_remote_copy(
            src_ref=src, dst_ref=dst, send_sem=ssem, recv_sem=rsem,
            device_id=(dev,), device_id_type=pl.DeviceIdType.MESH,
        )

    for h in range(4):
        s, d = h % 3, (h + 1) % 3
        if h == 0:
            srcR, dstR = commR_ref.at[0], commR_ref.at[1]
            srcL, dstL = commR_ref.at[0], commL_ref.at[1]
        elif h == 3:
            srcR, dstR = commR_ref.at[s, :, :HH], commR_ref.at[d, :, :HH]
            srcL, dstL = commL_ref.at[s, :, HH:], commL_ref.at[d, :, HH:]
        else:
            srcR, dstR = commR_ref.at[s], commR_ref.at[d]
            srcL, dstL = commL_ref.at[s], commL_ref.at[d]
        rR = mk_rdma(srcR, dstR, sendR_sems.at[s], recvR_sems.at[d], right)
        rL = mk_rdma(srcL, dstL, sendL_sems.at[s], recvL_sems.at[d], left)
        rR.start()
        rL.start()
        chunk_update(commR_ref, h % 3)
        if h > 0:
            chunk_update(commL_ref, h % 3)
        rR.wait()
        rL.wait()
    chunk_update(commR_ref, 1, 0, HH)
    chunk_update(commL_ref, 1, HH, H)

    def final_body(h, carry):
        out_ref[h] = acc_ref[h] / l_ref[h]
        return carry
    lax.fori_loop(0, H, final_body, 0)


def _perm(a):
    return a.reshape(R, R, 64, *a.shape[1:]).swapaxes(0, 1).reshape(a.shape)


def kernel(x, Wq, K_ext, V_ext, Wo):
    x2 = x.reshape(S, H * D)
    q = (jnp.dot(x2, Wq) * SCALE).reshape(S, H, D)
    qh = _perm(q).transpose(1, 0, 2).astype(jnp.bfloat16)
    kh = _perm(K_ext[0]).transpose(1, 0, 2).astype(jnp.bfloat16)
    vh = _perm(V_ext[0]).transpose(1, 0, 2).astype(jnp.bfloat16)

    ctx = pl.pallas_call(
        _attn_body,
        out_shape=jax.ShapeDtypeStruct((H, S, D), jnp.float32),
        in_specs=[
            pl.BlockSpec(memory_space=pltpu.VMEM),
            pl.BlockSpec(memory_space=pltpu.VMEM),
            pl.BlockSpec(memory_space=pltpu.VMEM),
        ],
        out_specs=pl.BlockSpec(memory_space=pltpu.VMEM),
        scratch_shapes=[
            pltpu.VMEM((3, 2, H, S, D), jnp.bfloat16),
            pltpu.VMEM((3, 2, H, S, D), jnp.bfloat16),
            pltpu.VMEM((H, S, D), jnp.float32),
            pltpu.VMEM((H, S, 1), jnp.float32),
            pltpu.VMEM((H, S, 1), jnp.float32),
            pltpu.SemaphoreType.DMA((3,)),
            pltpu.SemaphoreType.DMA((3,)),
            pltpu.SemaphoreType.DMA((3,)),
            pltpu.SemaphoreType.DMA((3,)),
        ],
        compiler_params=pltpu.CompilerParams(
            collective_id=0,
            vmem_limit_bytes=56 * 1024 * 1024,
        ),
    )(qh, kh, vh)

    ctx2 = _perm(ctx.transpose(1, 0, 2)).reshape(S, H * D)
    return jnp.dot(ctx2, Wo).reshape(1, S, H * D)


# device time: 231286 ns/iter; 2.2442x vs baseline; 1.0169x over previous
import jax
import jax.numpy as jnp
from jax import lax
from jax.experimental import pallas as pl
from jax.experimental.pallas import tpu as pltpu

N_DEV = 8
S = 1024
H = 8
D = 128
R = 4
RS = S // R
SCALE = 0.08838834764831843


def _attn_body(q_ref, k_ref, v_ref, out_ref,
               commR_ref, commL_ref, acc_ref, m_ref, l_ref,
               sendRA_sems, recvRA_sems, sendRB_sems, recvRB_sems,
               sendLA_sems, recvLA_sems, sendLB_sems, recvLB_sems):
    my = lax.axis_index("i")
    left = lax.rem(my - 1 + N_DEV, N_DEV)
    right = lax.rem(my + 1, N_DEV)

    barrier_sem = pltpu.get_barrier_semaphore()
    for nbr in (left, right):
        pl.semaphore_signal(barrier_sem, inc=1, device_id=(nbr,),
                            device_id_type=pl.DeviceIdType.MESH)
    pl.semaphore_wait(barrier_sem, 2)

    m_ref[:, :, :] = jnp.full((H, S, 1), -1e30, jnp.float32)
    l_ref[:, :, :] = jnp.zeros((H, S, 1), jnp.float32)
    acc_ref[:, :, :] = jnp.zeros((H, S, D), jnp.float32)

    commR_ref[0, 0] = k_ref[:, :, :]
    commR_ref[0, 1] = v_ref[:, :, :]

    def chunk_update(comm_ref, slot, lo=0, hi=H):
        def head_body(h, carry):
            for r in range(R):
                rs = pl.ds(r * RS, RS)
                q_h = q_ref[h, rs]
                k_h = comm_ref[slot, 0, h, rs]
                v_h = comm_ref[slot, 1, h, rs]
                s = lax.dot_general(
                    q_h, k_h, (((1,), (1,)), ((), ())),
                    preferred_element_type=jnp.float32,
                )
                m_old = m_ref[h, rs]
                m_new = jnp.maximum(m_old, jnp.max(s, axis=1, keepdims=True))
                alpha = jnp.exp(m_old - m_new)
                p = jnp.exp(s - m_new)
                l_ref[h, rs] = l_ref[h, rs] * alpha + jnp.sum(
                    p, axis=1, keepdims=True)
                pv = lax.dot_general(
                    p.astype(jnp.bfloat16), v_h, (((1,), (0,)), ((), ())),
                    preferred_element_type=jnp.float32,
                )
                acc_ref[h, rs] = acc_ref[h, rs] * alpha + pv
                m_ref[h, rs] = m_new
            return carry
        lax.fori_loop(lo, hi, head_body, 0)

    HH = H // 2
    HALF = (slice(0, HH), slice(HH, H))

    def half_rdma(src_ref, s_slot, dst_ref, d_slot, half, ssem, rsem, dev):
        return pltpu.make_async_remote_copy(
            src_ref=src_ref.at[s_slot, :, HALF[half]],
            dst_ref=dst_ref.at[d_slot, :, HALF[half]],
            send_sem=ssem, recv_sem=rsem,
            device_id=(dev,), device_id_type=pl.DeviceIdType.MESH,
        )

    def start_R(h, half):
        sems = (sendRA_sems, recvRA_sems) if half == 0 else (
            sendRB_sems, recvRB_sems)
        r = half_rdma(commR_ref, h % 3, commR_ref, (h + 1) % 3, half,
                      sems[0].at[h % 3], sems[1].at[(h + 1) % 3], right)
        r.start()
        return r

    def start_L(h, half):
        sems = (sendLA_sems, recvLA_sems) if half == 0 else (
            sendLB_sems, recvLB_sems)
        src = commR_ref if h == 0 else commL_ref
        r = half_rdma(src, h % 3, commL_ref, (h + 1) % 3, half,
                      sems[0].at[h % 3], sems[1].at[(h + 1) % 3], left)
        r.start()
        return r

    rRA, rRB = start_R(0, 0), start_R(0, 1)
    rLA, rLB = start_L(0, 0), start_L(0, 1)
    for h in range(4):
        chunk_update(commR_ref, h % 3)
        if h > 0:
            chunk_update(commL_ref, h % 3)
        rRA.wait()
        if h < 3:
            rRA = start_R(h + 1, 0)
        rLB.wait()
        if h < 3:
            rLB = start_L(h + 1, 1)
        if h < 3:
            rRB.wait()
            if h < 2:
                rRB = start_R(h + 1, 1)
            rLA.wait()
            if h < 2:
                rLA = start_L(h + 1, 0)
    chunk_update(commR_ref, 1, 0, HH)
    chunk_update(commL_ref, 1, HH, H)

    def final_body(h, carry):
        out_ref[h] = acc_ref[h] / l_ref[h]
        return carry
    lax.fori_loop(0, H, final_body, 0)


def _perm(a):
    return a.reshape(R, R, 64, *a.shape[1:]).swapaxes(0, 1).reshape(a.shape)


def kernel(x, Wq, K_ext, V_ext, Wo):
    x2 = x.reshape(S, H * D)
    q = (jnp.dot(x2, Wq) * SCALE).reshape(S, H, D)
    qh = _perm(q).transpose(1, 0, 2).astype(jnp.bfloat16)
    kh = _perm(K_ext[0]).transpose(1, 0, 2).astype(jnp.bfloat16)
    vh = _perm(V_ext[0]).transpose(1, 0, 2).astype(jnp.bfloat16)

    ctx = pl.pallas_call(
        _attn_body,
        out_shape=jax.ShapeDtypeStruct((H, S, D), jnp.float32),
        in_specs=[
            pl.BlockSpec(memory_space=pltpu.VMEM),
            pl.BlockSpec(memory_space=pltpu.VMEM),
            pl.BlockSpec(memory_space=pltpu.VMEM),
        ],
        out_specs=pl.BlockSpec(memory_space=pltpu.VMEM),
        scratch_shapes=[
            pltpu.VMEM((3, 2, H, S, D), jnp.bfloat16),
            pltpu.VMEM((3, 2, H, S, D), jnp.bfloat16),
            pltpu.VMEM((H, S, D), jnp.float32),
            pltpu.VMEM((H, S, 1), jnp.float32),
            pltpu.VMEM((H, S, 1), jnp.float32),
            pltpu.SemaphoreType.DMA((3,)),
            pltpu.SemaphoreType.DMA((3,)),
            pltpu.SemaphoreType.DMA((3,)),
            pltpu.SemaphoreType.DMA((3,)),
            pltpu.SemaphoreType.DMA((3,)),
            pltpu.SemaphoreType.DMA((3,)),
            pltpu.SemaphoreType.DMA((3,)),
            pltpu.SemaphoreType.DMA((3,)),
        ],
        compiler_params=pltpu.CompilerParams(
            collective_id=0,
            vmem_limit_bytes=56 * 1024 * 1024,
        ),
    )(qh, kh, vh)

    ctx2 = _perm(ctx.transpose(1, 0, 2)).reshape(S, H * D)
    return jnp.dot(ctx2, Wo).reshape(1, S, H * D)


# device time: 197885 ns/iter; 2.6229x vs baseline; 1.1688x over previous
import jax
import jax.numpy as jnp
from jax import lax
from jax.experimental import pallas as pl
from jax.experimental.pallas import tpu as pltpu

N_DEV = 8
S = 1024
H = 8
D = 128
R = 4
RS = S // R
G = H * R
GH = G // 2
SCALE = 0.08838834764831843


def _attn_body(q_ref, k_ref, v_ref, out_ref,
               commR_ref, commL_ref, l_ref,
               sendRA_sems, recvRA_sems, sendRB_sems, recvRB_sems,
               sendLA_sems, recvLA_sems, sendLB_sems, recvLB_sems):
    my = lax.axis_index("i")
    left = lax.rem(my - 1 + N_DEV, N_DEV)
    right = lax.rem(my + 1, N_DEV)

    barrier_sem = pltpu.get_barrier_semaphore()
    for nbr in (left, right):
        pl.semaphore_signal(barrier_sem, inc=1, device_id=(nbr,),
                            device_id_type=pl.DeviceIdType.MESH)
    pl.semaphore_wait(barrier_sem, 2)

    l_ref[:, :, :] = jnp.zeros((G, RS, 1), jnp.float32)
    out_ref[:, :, :] = jnp.zeros((G, RS, D), jnp.float32)

    commR_ref[0, 0] = k_ref[:, :, :]
    commR_ref[0, 1] = v_ref[:, :, :]

    def chunk_update(comm_ref, slot):
        half_update(comm_ref, slot, 0, GH)
        half_update(comm_ref, slot, GH, G)

    def half_update(comm_ref, slot, lo, hi):
        gs = slice(lo, hi)
        q = q_ref[gs]
        k = comm_ref[slot, 0, gs]
        v = comm_ref[slot, 1, gs]
        s = lax.dot_general(
            q, k, (((2,), (2,)), ((0,), (0,))),
            preferred_element_type=jnp.float32,
        )
        p = jnp.exp(s)
        l_ref[gs] = l_ref[gs] + jnp.sum(p, axis=2, keepdims=True)
        pv = lax.dot_general(
            p.astype(jnp.bfloat16), v, (((2,), (1,)), ((0,), (0,))),
            preferred_element_type=jnp.float32,
        )
        out_ref[gs] = out_ref[gs] + pv

    HALF = (slice(0, GH), slice(GH, G))

    def half_rdma(src_ref, s_slot, dst_ref, d_slot, half, ssem, rsem, dev):
        return pltpu.make_async_remote_copy(
            src_ref=src_ref.at[s_slot, :, HALF[half]],
            dst_ref=dst_ref.at[d_slot, :, HALF[half]],
            send_sem=ssem, recv_sem=rsem,
            device_id=(dev,), device_id_type=pl.DeviceIdType.MESH,
        )

    def start_R(h, half):
        sems = (sendRA_sems, recvRA_sems) if half == 0 else (
            sendRB_sems, recvRB_sems)
        r = half_rdma(commR_ref, h % 3, commR_ref, (h + 1) % 3, half,
                      sems[0].at[h % 3], sems[1].at[(h + 1) % 3], right)
        r.start()
        return r

    def start_L(h, half):
        sems = (sendLA_sems, recvLA_sems) if half == 0 else (
            sendLB_sems, recvLB_sems)
        src = commR_ref if h == 0 else commL_ref
        r = half_rdma(src, h % 3, commL_ref, (h + 1) % 3, half,
                      sems[0].at[h % 3], sems[1].at[(h + 1) % 3], left)
        r.start()
        return r

    rRA, rRB = start_R(0, 0), start_R(0, 1)
    rLA, rLB = start_L(0, 0), start_L(0, 1)
    for h in range(4):
        chunk_update(commR_ref, h % 3)
        if h > 0:
            chunk_update(commL_ref, h % 3)
        rRA.wait()
        if h < 3:
            rRA = start_R(h + 1, 0)
        rLB.wait()
        if h < 3:
            rLB = start_L(h + 1, 1)
        if h < 3:
            rRB.wait()
            if h < 2:
                rRB = start_R(h + 1, 1)
            rLA.wait()
            if h < 2:
                rLA = start_L(h + 1, 0)
    half_update(commR_ref, 1, 0, GH)
    half_update(commL_ref, 1, GH, G)

    out_ref[:, :, :] = out_ref[:, :, :] / l_ref[:, :, :]


def _perm(a):
    return a.reshape(R, R, 64, *a.shape[1:]).swapaxes(0, 1).reshape(a.shape)


def _to_tiles(a):
    return _perm(a).transpose(1, 0, 2).reshape(G, RS, D)


def kernel(x, Wq, K_ext, V_ext, Wo):
    x2 = x.reshape(S, H * D)
    q = (jnp.dot(x2, Wq) * SCALE).reshape(S, H, D)
    qh = _to_tiles(q).astype(jnp.bfloat16)
    kh = _to_tiles(K_ext[0]).astype(jnp.bfloat16)
    vh = _to_tiles(V_ext[0]).astype(jnp.bfloat16)

    ctx = pl.pallas_call(
        _attn_body,
        out_shape=jax.ShapeDtypeStruct((G, RS, D), jnp.float32),
        in_specs=[
            pl.BlockSpec(memory_space=pltpu.VMEM),
            pl.BlockSpec(memory_space=pltpu.VMEM),
            pl.BlockSpec(memory_space=pltpu.VMEM),
        ],
        out_specs=pl.BlockSpec(memory_space=pltpu.VMEM),
        scratch_shapes=[
            pltpu.VMEM((3, 2, G, RS, D), jnp.bfloat16),
            pltpu.VMEM((3, 2, G, RS, D), jnp.bfloat16),
            pltpu.VMEM((G, RS, 1), jnp.float32),
            pltpu.SemaphoreType.DMA((3,)),
            pltpu.SemaphoreType.DMA((3,)),
            pltpu.SemaphoreType.DMA((3,)),
            pltpu.SemaphoreType.DMA((3,)),
            pltpu.SemaphoreType.DMA((3,)),
            pltpu.SemaphoreType.DMA((3,)),
            pltpu.SemaphoreType.DMA((3,)),
            pltpu.SemaphoreType.DMA((3,)),
        ],
        compiler_params=pltpu.CompilerParams(
            collective_id=0,
            vmem_limit_bytes=56 * 1024 * 1024,
        ),
    )(qh, kh, vh)

    ctx2 = _perm(ctx.reshape(H, S, D).transpose(1, 0, 2)).reshape(S, H * D)
    return jnp.dot(ctx2, Wo).reshape(1, S, H * D)
